# baseline (speedup 1.0000x reference)
import functools

import jax
import jax.numpy as jnp
from jax import lax
from jax.experimental import pallas as pl
from jax.experimental.pallas import tpu as pltpu
from jax.experimental.pallas import tpu_sc as plsc

_B = 16384
_C = 100
_ES = 10
_MOM = 0.9


def _sc_gather(table, idx):
  info = plsc.get_sparse_core_info()
  nw = info.num_cores * info.num_subcores
  b_per_w = _B // nw
  n_sem = 8
  mesh = plsc.VectorSubcoreMesh(core_axis_name="c", subcore_axis_name="s")

  @functools.partial(
      pl.kernel,
      mesh=mesh,
      out_type=jax.ShapeDtypeStruct((_B, _C), jnp.float32),
      scratch_types=[
          pltpu.VMEM((b_per_w,), jnp.int32),
          pltpu.VMEM((b_per_w, _C), jnp.float32),
          [pltpu.SemaphoreType.DMA for _ in range(n_sem)],
      ],
      compiler_params=pltpu.CompilerParams(needs_layout_passes=False),
  )
  def k(table_hbm, idx_hbm, out_hbm, idx_v, rows_v, sems):
    wid = lax.axis_index("s") * info.num_cores + lax.axis_index("c")
    base = wid * b_per_w
    lane = lax.broadcasted_iota(jnp.int32, (16,), 0)
    pltpu.sync_copy(idx_hbm.at[pl.ds(base, b_per_w)], idx_v)
    descs = []
    for v in range(b_per_w // 16):
      vec = idx_v[pl.ds(v * 16, 16)]
      for l in range(16):
        sc = jnp.sum(jnp.where(lane == l, vec, 0))
        r = v * 16 + l
        descs.append(pltpu.async_copy(table_hbm.at[pl.ds(sc, 1)],
                                      rows_v.at[pl.ds(r, 1)],
                                      sems[r % n_sem]))
    for d in descs:
      d.wait()
    pltpu.sync_copy(rows_v, out_hbm.at[pl.ds(base, b_per_w)])

  return k(table, idx)


def _tc_loss(epoch_s, logits, labels3, g):
  blk = 1024
  grid = _B // blk

  def body(epoch_ref, x_ref, lbl_ref, g_ref, out_ref, acc_ref):
    i = pl.program_id(0)

    @pl.when(i == 0)
    def _init():
      acc_ref[0] = 0.0
      acc_ref[1] = 0.0
      acc_ref[2] = 0.0

    x = x_ref[...]
    m = jnp.max(x, axis=1, keepdims=True)
    ex = jnp.exp(x - m)
    s = jnp.sum(ex, axis=1, keepdims=True)
    log_pred = x - m - jnp.log(s)
    pred = ex / s
    lbl = lbl_ref[0, 0, :]
    cols = lax.broadcasted_iota(jnp.int32, (blk, _C), 1)
    onehot = cols == lbl[:, None]
    g = g_ref[...]
    acc_ref[0] += jnp.sum(jnp.where(onehot, log_pred, 0.0))
    acc_ref[1] += jnp.sum(log_pred * g)
    acc_ref[2] += jnp.sum(log_pred * pred)

    @pl.when(i == grid - 1)
    def _fin():
      ce = -acc_ref[0] / _B
      selc = -(_MOM * acc_ref[1] + (1.0 - _MOM) * acc_ref[2]) / _B
      out_ref[0, 0] = jnp.where(epoch_ref[0, 0] <= _ES, ce, selc)

  return pl.pallas_call(
      body,
      grid=(grid,),
      in_specs=[
          pl.BlockSpec(memory_space=pltpu.SMEM),
          pl.BlockSpec((blk, _C), lambda i: (i, 0)),
          pl.BlockSpec((1, 1, blk), lambda i: (i, 0, 0)),
          pl.BlockSpec((blk, _C), lambda i: (i, 0)),
      ],
      out_specs=pl.BlockSpec(memory_space=pltpu.SMEM),
      out_shape=jax.ShapeDtypeStruct((1, 1), jnp.float32),
      scratch_shapes=[pltpu.SMEM((3,), jnp.float32)],
  )(epoch_s, logits, labels3, g)


def kernel(logits, labels, index, epoch, soft_labels):
  g = _sc_gather(soft_labels, index)
  labels3 = labels.astype(jnp.int32).reshape(_B // 1024, 1, 1024)
  epoch_s = jnp.asarray(epoch, jnp.int32).reshape(1, 1)
  out = _tc_loss(epoch_s, logits, labels3, g)
  return out[0, 0]
